# initial kernel scaffold (unmeasured)
import jax
import jax.numpy as jnp
from jax import lax
from jax.experimental import pallas as pl
from jax.experimental.pallas import tpu as pltpu

N_DEV = 4
SQ = 256
DH = 128
H_PER = 8
H_TOT = 32
SKV_PER = 4096
D_MODEL = 1024
BLK = 64
SCALE = 0.08838834764831843


def _peers(me):
    return [(me + o) % N_DEV for o in range(1, N_DEV)]


def _barrier_all():
    me = lax.axis_index("i")
    sem = pltpu.get_barrier_semaphore()
    for peer in _peers(me):
        pl.semaphore_signal(
            sem, inc=1, device_id=(peer,), device_id_type=pl.DeviceIdType.MESH
        )
    pl.semaphore_wait(sem, N_DEV - 1)


def _qall_body(x_ref, wq_ref, qall_ref, send_sems, recv_sems):
    me = lax.axis_index("i")
    _barrier_all()

    q = jnp.dot(x_ref[0], wq_ref[...], preferred_element_type=jnp.float32)
    qall_ref[pl.ds(me, 1)] = q[None]

    sends = []
    for o in range(1, N_DEV):
        peer = (me + o) % N_DEV
        rdma = pltpu.make_async_remote_copy(
            src_ref=qall_ref.at[me],
            dst_ref=qall_ref.at[me],
            send_sem=send_sems.at[o - 1],
            recv_sem=recv_sems.at[me],
            device_id=(peer,),
            device_id_type=pl.DeviceIdType.MESH,
        )
        rdma.start()
        sends.append(rdma)
    for o in range(1, N_DEV):
        j = (me + o) % N_DEV
        pltpu.make_async_remote_copy(
            src_ref=qall_ref.at[me],
            dst_ref=qall_ref.at[j],
            send_sem=send_sems.at[o - 1],
            recv_sem=recv_sems.at[j],
            device_id=(me,),
            device_id_type=pl.DeviceIdType.MESH,
        ).wait_recv()
    for rdma in sends:
        rdma.wait_send()


def _attn_body(q_ref, k_ref, v_ref, ctx_ref, stats_ref):
    me = lax.axis_index("i")
    q = q_ref[0, :, 0, :]
    k = k_ref[0, :, 0, :]
    v = v_ref[0, :, 0, :]

    scores = (
        lax.dot_general(
            q, k, (((1,), (1,)), ((), ())), preferred_element_type=jnp.float32
        )
        * SCALE
    )
    qb = lax.broadcasted_iota(jnp.int32, (SQ, SKV_PER), 0) // BLK
    kb = lax.broadcasted_iota(jnp.int32, (SQ, SKV_PER), 1) // BLK + (
        SKV_PER // BLK
    ) * me
    mask = (qb == kb) | (kb == 0) | ((qb + kb) % 3 == 0)
    scores = jnp.where(mask, scores, -1e9)
    m = jnp.max(scores, axis=1, keepdims=True)
    p = jnp.exp(scores - m)
    s = jnp.sum(p, axis=1, keepdims=True)
    ctx_ref[0, :, 0, :] = jnp.dot(p, v, preferred_element_type=jnp.float32)
    stats_ref[0, 0] = m
    stats_ref[0, 1] = s


def _combine_body(
    ctx_ref,
    stats_ref,
    wo_ref,
    out_ref,
    rctx,
    rstats,
    ry,
    ctx_send_sems,
    stats_send_sems,
    y_send_sems,
    ctx_recv_sems,
    stats_recv_sems,
    y_recv_sems,
):
    me = lax.axis_index("i")
    _barrier_all()

    sends = []
    for o in range(1, N_DEV):
        peer = (me + o) % N_DEV
        c = pltpu.make_async_remote_copy(
            src_ref=ctx_ref.at[peer],
            dst_ref=rctx.at[me],
            send_sem=ctx_send_sems.at[o - 1],
            recv_sem=ctx_recv_sems.at[me],
            device_id=(peer,),
            device_id_type=pl.DeviceIdType.MESH,
        )
        c.start()
        st = pltpu.make_async_remote_copy(
            src_ref=stats_ref.at[peer],
            dst_ref=rstats.at[me],
            send_sem=stats_send_sems.at[o - 1],
            recv_sem=stats_recv_sems.at[me],
            device_id=(peer,),
            device_id_type=pl.DeviceIdType.MESH,
        )
        st.start()
        sends += [c, st]

    rctx[pl.ds(me, 1)] = ctx_ref[pl.ds(me, 1)]
    rstats[pl.ds(me, 1)] = stats_ref[pl.ds(me, 1)]

    for o in range(1, N_DEV):
        j = (me + o) % N_DEV
        pltpu.make_async_remote_copy(
            src_ref=ctx_ref.at[me],
            dst_ref=rctx.at[j],
            send_sem=ctx_send_sems.at[o - 1],
            recv_sem=ctx_recv_sems.at[j],
            device_id=(me,),
            device_id_type=pl.DeviceIdType.MESH,
        ).wait_recv()
        pltpu.make_async_remote_copy(
            src_ref=stats_ref.at[me],
            dst_ref=rstats.at[j],
            send_sem=stats_send_sems.at[o - 1],
            recv_sem=stats_recv_sems.at[j],
            device_id=(me,),
            device_id_type=pl.DeviceIdType.MESH,
        ).wait_recv()

    m_all = rstats[:, 0]
    s_all = rstats[:, 1]
    mmax = jnp.max(m_all, axis=0)
    coef = jnp.exp(m_all - mmax[None])
    den = jnp.sum(coef * s_all, axis=0)
    num = jnp.sum(coef[..., None] * rctx[...], axis=0)
    ctx = num / den[..., None]

    y = jnp.zeros((SQ, D_MODEL), jnp.float32)
    for h in range(H_PER):
        y = y + jnp.dot(
            ctx[:, h, :],
            wo_ref[h * DH : (h + 1) * DH, :],
            preferred_element_type=jnp.float32,
        )

    ry[pl.ds(me, 1)] = y[None]
    ysends = []
    for o in range(1, N_DEV):
        peer = (me + o) % N_DEV
        r = pltpu.make_async_remote_copy(
            src_ref=ry.at[me],
            dst_ref=ry.at[me],
            send_sem=y_send_sems.at[o - 1],
            recv_sem=y_recv_sems.at[me],
            device_id=(peer,),
            device_id_type=pl.DeviceIdType.MESH,
        )
        r.start()
        ysends.append(r)
    for o in range(1, N_DEV):
        j = (me + o) % N_DEV
        pltpu.make_async_remote_copy(
            src_ref=ry.at[me],
            dst_ref=ry.at[j],
            send_sem=y_send_sems.at[o - 1],
            recv_sem=y_recv_sems.at[j],
            device_id=(me,),
            device_id_type=pl.DeviceIdType.MESH,
        ).wait_recv()

    out_ref[0] = ry[0] + ry[1] + ry[2] + ry[3]
    for rdma in sends + ysends:
        rdma.wait_send()


def kernel(x, Wq, K_ext, V_ext, Wo):
    qall = pl.pallas_call(
        _qall_body,
        out_shape=jax.ShapeDtypeStruct((N_DEV, SQ, D_MODEL), jnp.float32),
        in_specs=[
            pl.BlockSpec(memory_space=pltpu.VMEM),
            pl.BlockSpec(memory_space=pltpu.VMEM),
        ],
        out_specs=pl.BlockSpec(memory_space=pltpu.VMEM),
        scratch_shapes=[
            pltpu.SemaphoreType.DMA((N_DEV - 1,)),
            pltpu.SemaphoreType.DMA((N_DEV,)),
        ],
        compiler_params=pltpu.CompilerParams(collective_id=0),
    )(x, Wq)

    qheads = qall.reshape(N_DEV, SQ, H_PER, DH)

    ctx, stats = pl.pallas_call(
        _attn_body,
        grid=(H_TOT,),
        in_specs=[
            pl.BlockSpec((1, SQ, 1, DH), lambda h: (h // H_PER, 0, h % H_PER, 0)),
            pl.BlockSpec((1, SKV_PER, 1, DH), lambda h: (0, 0, h, 0)),
            pl.BlockSpec((1, SKV_PER, 1, DH), lambda h: (0, 0, h, 0)),
        ],
        out_shape=[
            jax.ShapeDtypeStruct((N_DEV, SQ, H_PER, DH), jnp.float32),
            jax.ShapeDtypeStruct((N_DEV, 2, SQ, H_PER), jnp.float32),
        ],
        out_specs=[
            pl.BlockSpec((1, SQ, 1, DH), lambda h: (h // H_PER, 0, h % H_PER, 0)),
            pl.BlockSpec((1, 2, SQ, 1), lambda h: (h // H_PER, 0, 0, h % H_PER)),
        ],
    )(qheads, K_ext, V_ext)

    return pl.pallas_call(
        _combine_body,
        out_shape=jax.ShapeDtypeStruct((1, SQ, D_MODEL), jnp.float32),
        in_specs=[
            pl.BlockSpec(memory_space=pltpu.VMEM),
            pl.BlockSpec(memory_space=pltpu.VMEM),
            pl.BlockSpec(memory_space=pltpu.VMEM),
        ],
        out_specs=pl.BlockSpec(memory_space=pltpu.VMEM),
        scratch_shapes=[
            pltpu.VMEM((N_DEV, SQ, H_PER, DH), jnp.float32),
            pltpu.VMEM((N_DEV, 2, SQ, H_PER), jnp.float32),
            pltpu.VMEM((N_DEV, SQ, D_MODEL), jnp.float32),
            pltpu.SemaphoreType.DMA((N_DEV - 1,)),
            pltpu.SemaphoreType.DMA((N_DEV - 1,)),
            pltpu.SemaphoreType.DMA((N_DEV - 1,)),
            pltpu.SemaphoreType.DMA((N_DEV,)),
            pltpu.SemaphoreType.DMA((N_DEV,)),
            pltpu.SemaphoreType.DMA((N_DEV,)),
        ],
        compiler_params=pltpu.CompilerParams(collective_id=1),
    )(ctx, stats, Wo)


# baseline (device time: 414504 ns/iter reference)
import jax
import jax.numpy as jnp
from jax import lax
from jax.experimental import pallas as pl
from jax.experimental.pallas import tpu as pltpu

N_DEV = 4
SQ = 256
DH = 128
H_PER = 8
H_TOT = 32
SKV_PER = 4096
D_MODEL = 1024
BLK = 64
SCALE = 0.08838834764831843


def _peers(me):
    return [(me + o) % N_DEV for o in range(1, N_DEV)]


def _barrier_all():
    me = lax.axis_index("i")
    sem = pltpu.get_barrier_semaphore()
    for peer in _peers(me):
        pl.semaphore_signal(
            sem, inc=1, device_id=(peer,), device_id_type=pl.DeviceIdType.MESH
        )
    pl.semaphore_wait(sem, N_DEV - 1)


def _qall_body(x_ref, wq_ref, qall_ref, send_sems, recv_sems):
    me = lax.axis_index("i")
    _barrier_all()

    q = jnp.dot(x_ref[0], wq_ref[...], preferred_element_type=jnp.float32)
    qall_ref[pl.ds(me, 1)] = q[None]

    sends = []
    for o in range(1, N_DEV):
        peer = (me + o) % N_DEV
        rdma = pltpu.make_async_remote_copy(
            src_ref=qall_ref.at[me],
            dst_ref=qall_ref.at[me],
            send_sem=send_sems.at[o - 1],
            recv_sem=recv_sems.at[me],
            device_id=(peer,),
            device_id_type=pl.DeviceIdType.MESH,
        )
        rdma.start()
        sends.append(rdma)
    for o in range(1, N_DEV):
        j = (me + o) % N_DEV
        pltpu.make_async_remote_copy(
            src_ref=qall_ref.at[me],
            dst_ref=qall_ref.at[j],
            send_sem=send_sems.at[o - 1],
            recv_sem=recv_sems.at[j],
            device_id=(me,),
            device_id_type=pl.DeviceIdType.MESH,
        ).wait_recv()
    for rdma in sends:
        rdma.wait_send()


def _attn_body(q_ref, k_ref, v_ref, ctx_ref, stats_ref):
    me = lax.axis_index("i")
    q = q_ref[0]
    k = k_ref[0]
    v = v_ref[0]

    scores = (
        lax.dot_general(
            q, k, (((1,), (1,)), ((), ())), preferred_element_type=jnp.float32
        )
        * SCALE
    )
    qb = lax.broadcasted_iota(jnp.int32, (SQ, SKV_PER), 0) // BLK
    kb = lax.broadcasted_iota(jnp.int32, (SQ, SKV_PER), 1) // BLK + (
        SKV_PER // BLK
    ) * me
    mask = (qb == kb) | (kb == 0) | ((qb + kb) % 3 == 0)
    scores = jnp.where(mask, scores, -1e9)
    m = jnp.max(scores, axis=1, keepdims=True)
    p = jnp.exp(scores - m)
    s = jnp.sum(p, axis=1, keepdims=True)
    ctx_ref[0] = jnp.dot(p, v, preferred_element_type=jnp.float32)
    stats_ref[0, 0] = jnp.concatenate([m, s], axis=1)


def _combine_body(
    ctx_ref,
    stats_ref,
    wo_ref,
    out_ref,
    rctx,
    rstats,
    ry,
    ctx_send_sems,
    stats_send_sems,
    y_send_sems,
    ctx_recv_sems,
    stats_recv_sems,
    y_recv_sems,
):
    me = lax.axis_index("i")
    _barrier_all()

    sends = []
    for o in range(1, N_DEV):
        peer = (me + o) % N_DEV
        c = pltpu.make_async_remote_copy(
            src_ref=ctx_ref.at[peer],
            dst_ref=rctx.at[me],
            send_sem=ctx_send_sems.at[o - 1],
            recv_sem=ctx_recv_sems.at[me],
            device_id=(peer,),
            device_id_type=pl.DeviceIdType.MESH,
        )
        c.start()
        st = pltpu.make_async_remote_copy(
            src_ref=stats_ref.at[peer],
            dst_ref=rstats.at[me],
            send_sem=stats_send_sems.at[o - 1],
            recv_sem=stats_recv_sems.at[me],
            device_id=(peer,),
            device_id_type=pl.DeviceIdType.MESH,
        )
        st.start()
        sends += [c, st]

    rctx[pl.ds(me, 1)] = ctx_ref[pl.ds(me, 1)]
    rstats[pl.ds(me, 1)] = stats_ref[pl.ds(me, 1)]

    for o in range(1, N_DEV):
        j = (me + o) % N_DEV
        pltpu.make_async_remote_copy(
            src_ref=ctx_ref.at[me],
            dst_ref=rctx.at[j],
            send_sem=ctx_send_sems.at[o - 1],
            recv_sem=ctx_recv_sems.at[j],
            device_id=(me,),
            device_id_type=pl.DeviceIdType.MESH,
        ).wait_recv()
        pltpu.make_async_remote_copy(
            src_ref=stats_ref.at[me],
            dst_ref=rstats.at[j],
            send_sem=stats_send_sems.at[o - 1],
            recv_sem=stats_recv_sems.at[j],
            device_id=(me,),
            device_id_type=pl.DeviceIdType.MESH,
        ).wait_recv()

    y = jnp.zeros((SQ, D_MODEL), jnp.float32)
    for h in range(H_PER):
        m_h = rstats[:, h, :, 0]
        s_h = rstats[:, h, :, 1]
        mmax = jnp.max(m_h, axis=0)
        coef = jnp.exp(m_h - mmax[None])
        den = jnp.sum(coef * s_h, axis=0)
        num = jnp.sum(
            coef[:, :, None] * rctx[:, :, h * DH : (h + 1) * DH], axis=0
        )
        ctx_h = num / den[:, None]
        y = y + jnp.dot(
            ctx_h,
            wo_ref[h * DH : (h + 1) * DH, :],
            preferred_element_type=jnp.float32,
        )

    ry[pl.ds(me, 1)] = y[None]
    ysends = []
    for o in range(1, N_DEV):
        peer = (me + o) % N_DEV
        r = pltpu.make_async_remote_copy(
            src_ref=ry.at[me],
            dst_ref=ry.at[me],
            send_sem=y_send_sems.at[o - 1],
            recv_sem=y_recv_sems.at[me],
            device_id=(peer,),
            device_id_type=pl.DeviceIdType.MESH,
        )
        r.start()
        ysends.append(r)
    for o in range(1, N_DEV):
        j = (me + o) % N_DEV
        pltpu.make_async_remote_copy(
            src_ref=ry.at[me],
            dst_ref=ry.at[j],
            send_sem=y_send_sems.at[o - 1],
            recv_sem=y_recv_sems.at[j],
            device_id=(me,),
            device_id_type=pl.DeviceIdType.MESH,
        ).wait_recv()

    out_ref[0] = ry[0] + ry[1] + ry[2] + ry[3]
    for rdma in sends + ysends:
        rdma.wait_send()


def kernel(x, Wq, K_ext, V_ext, Wo):
    qall = pl.pallas_call(
        _qall_body,
        out_shape=jax.ShapeDtypeStruct((N_DEV, SQ, D_MODEL), jnp.float32),
        in_specs=[
            pl.BlockSpec(memory_space=pltpu.VMEM),
            pl.BlockSpec(memory_space=pltpu.VMEM),
        ],
        out_specs=pl.BlockSpec(memory_space=pltpu.VMEM),
        scratch_shapes=[
            pltpu.SemaphoreType.DMA((N_DEV - 1,)),
            pltpu.SemaphoreType.DMA((N_DEV,)),
        ],
        compiler_params=pltpu.CompilerParams(collective_id=0),
    )(x, Wq)

    k_flat = K_ext.reshape(1, SKV_PER, H_TOT * DH)
    v_flat = V_ext.reshape(1, SKV_PER, H_TOT * DH)

    ctx, stats = pl.pallas_call(
        _attn_body,
        grid=(H_TOT,),
        in_specs=[
            pl.BlockSpec((1, SQ, DH), lambda h: (h // H_PER, 0, h % H_PER)),
            pl.BlockSpec((1, SKV_PER, DH), lambda h: (0, 0, h)),
            pl.BlockSpec((1, SKV_PER, DH), lambda h: (0, 0, h)),
        ],
        out_shape=[
            jax.ShapeDtypeStruct((N_DEV, SQ, H_PER * DH), jnp.float32),
            jax.ShapeDtypeStruct((N_DEV, H_PER, SQ, 2), jnp.float32),
        ],
        out_specs=[
            pl.BlockSpec((1, SQ, DH), lambda h: (h // H_PER, 0, h % H_PER)),
            pl.BlockSpec((1, 1, SQ, 2), lambda h: (h // H_PER, h % H_PER, 0, 0)),
        ],
    )(qall, k_flat, v_flat)

    return pl.pallas_call(
        _combine_body,
        out_shape=jax.ShapeDtypeStruct((1, SQ, D_MODEL), jnp.float32),
        in_specs=[
            pl.BlockSpec(memory_space=pltpu.VMEM),
            pl.BlockSpec(memory_space=pltpu.VMEM),
            pl.BlockSpec(memory_space=pltpu.VMEM),
        ],
        out_specs=pl.BlockSpec(memory_space=pltpu.VMEM),
        scratch_shapes=[
            pltpu.VMEM((N_DEV, SQ, H_PER * DH), jnp.float32),
            pltpu.VMEM((N_DEV, H_PER, SQ, 2), jnp.float32),
            pltpu.VMEM((N_DEV, SQ, D_MODEL), jnp.float32),
            pltpu.SemaphoreType.DMA((N_DEV - 1,)),
            pltpu.SemaphoreType.DMA((N_DEV - 1,)),
            pltpu.SemaphoreType.DMA((N_DEV - 1,)),
            pltpu.SemaphoreType.DMA((N_DEV,)),
            pltpu.SemaphoreType.DMA((N_DEV,)),
            pltpu.SemaphoreType.DMA((N_DEV,)),
        ],
        compiler_params=pltpu.CompilerParams(collective_id=1),
    )(ctx, stats, Wo)
